# baseline (speedup 1.0000x reference)
import functools

import jax
import jax.numpy as jnp
from jax import lax
from jax.experimental import pallas as pl
from jax.experimental.pallas import tpu as pltpu
from jax.experimental.pallas import tpu_sc as plsc

_CH = 64
_EPS = 1e-5
_L = 16
_NC = 2
_NS = 16
_NW = _NC * _NS
_CHUNK = 128
_K = _CH // _L


def _rsqrt(t):
    i = lax.bitcast_convert_type(t, jnp.int32)
    i = jnp.int32(0x5F3759DF) - (i >> 1)
    y = lax.bitcast_convert_type(i, jnp.float32)
    for _ in range(3):
        y = y * (1.5 - 0.5 * t * y * y)
    return y


def _body(nchunk, x_hbm, table_hbm, gamma_hbm, beta_hbm, out_hbm,
          idx_v, in0_v, in1_v, out0_v, out1_v, gam_v, bet_v,
          gsem0, gsem1, ssem0, ssem1):
    cid = lax.axis_index("c")
    sid = lax.axis_index("s")
    wid = sid * _NC + cid
    base = wid * (nchunk * _CHUNK)

    pltpu.sync_copy(x_hbm.at[wid], idx_v)
    pltpu.sync_copy(gamma_hbm, gam_v)
    pltpu.sync_copy(beta_hbm, bet_v)

    gam = [gam_v[pl.ds(k * _L, _L)] for k in range(_K)]
    bet = [bet_v[pl.ds(k * _L, _L)] for k in range(_K)]

    inb = (in0_v, in1_v)
    outb = (out0_v, out1_v)
    gsem = (gsem0, gsem1)
    ssem = (ssem0, ssem1)

    def gather(j, b):
        return pltpu.make_async_copy(table_hbm.at[idx_v.at[j]], inb[b], gsem[b])

    def store(j, b):
        dst = out_hbm.at[pl.ds(base + j * _CHUNK, _CHUNK)]
        return pltpu.make_async_copy(outb[b], dst, ssem[b])

    gather(0, 0).start()
    gather(1, 1).start()

    @pl.loop(0, nchunk, step=2)
    def _(jj):
        for b in range(2):
            j = jj + b
            gather(j, b).wait()

            @pl.when(j >= 2)
            def _():
                store(j - 2, b).wait()

            src, dst = inb[b], outb[b]

            @plsc.parallel_loop(0, _CHUNK, unroll=8)
            def _(r):
                v = [src[r, pl.ds(k * _L, _L)] for k in range(_K)]
                tot = jnp.sum(v[0] + v[1] + v[2] + v[3])
                sq = jnp.sum(v[0] * v[0] + v[1] * v[1]
                             + v[2] * v[2] + v[3] * v[3])
                mean = tot * (1.0 / _CH)
                var = sq * (1.0 / _CH) - mean * mean
                a = _rsqrt(var + _EPS)
                for k in range(_K):
                    c = gam[k] * a
                    d = bet[k] - mean * c
                    dst[r, pl.ds(k * _L, _L)] = v[k] * c + d

            store(j, b).start()

            @pl.when(j + 2 < nchunk)
            def _():
                gather(j + 2, b).start()

    store(nchunk - 2, 0).wait()
    store(nchunk - 1, 1).wait()


def kernel(x, table, gamma, beta):
    b_total = x.shape[0] * x.shape[1]
    rows_w = b_total // _NW
    nchunk = rows_w // _CHUNK
    xr = x.reshape(_NW, nchunk, _CHUNK)

    mesh = plsc.VectorSubcoreMesh(
        core_axis_name="c", subcore_axis_name="s",
        num_cores=_NC, num_subcores=_NS)

    run = pl.kernel(
        functools.partial(_body, nchunk),
        out_type=jax.ShapeDtypeStruct((b_total, _CH), jnp.float32),
        mesh=mesh,
        compiler_params=pltpu.CompilerParams(
            needs_layout_passes=False, use_tc_tiling_on_sc=False),
        scratch_types=[
            pltpu.VMEM((nchunk, _CHUNK), jnp.int32),
            pltpu.VMEM((_CHUNK, _CH), jnp.float32),
            pltpu.VMEM((_CHUNK, _CH), jnp.float32),
            pltpu.VMEM((_CHUNK, _CH), jnp.float32),
            pltpu.VMEM((_CHUNK, _CH), jnp.float32),
            pltpu.VMEM((_CH,), jnp.float32),
            pltpu.VMEM((_CH,), jnp.float32),
            pltpu.SemaphoreType.DMA,
            pltpu.SemaphoreType.DMA,
            pltpu.SemaphoreType.DMA,
            pltpu.SemaphoreType.DMA,
        ],
    )
    out = run(xr, table, gamma, beta)
    return out.reshape(x.shape + (_CH,))
